# Initial kernel scaffold; baseline (speedup 1.0000x reference)
import jax
import jax.numpy as jnp
from jax.experimental import pallas as pl


def kernel(x, edge_index, W, b, eps):
    raise NotImplementedError("write your pallas kernel here")


# speedup vs baseline: 2.6143x; 2.6143x over previous
import functools

import jax
import jax.numpy as jnp
from jax import lax
from jax.experimental import pallas as pl
from jax.experimental.pallas import tpu as pltpu
from jax.experimental.pallas import tpu_sc as plsc

N_NODES = 10000
N_EDGES = 160000
D = 256
DH = 128
NC = 2
NS = 16
PAD_ROWS = 8
ZROW = N_NODES
CHUNK = 128
E_PAD = 163840
EDGES_PER_TILE = E_PAD // NS
NCHUNK = EDGES_PER_TILE // CHUNK
N_PAD = 10240
ROWS_PER_TILE = N_PAD // NS


def _sc_agg_body(xcat_hbm, src2_hbm, dst_hbm, zeros_hbm, out_hbm,
                 src_v, dst_v, rows_v, agg_sh, sem):
    c = lax.axis_index("c")
    s = lax.axis_index("s")
    stripe = pl.ds(s * ROWS_PER_TILE, ROWS_PER_TILE)
    pltpu.sync_copy(zeros_hbm, agg_sh.at[stripe])
    plsc.subcore_barrier()

    ebase = s * EDGES_PER_TILE

    def chunk_body(k, carry):
        base = ebase + k * CHUNK
        pltpu.sync_copy(src2_hbm.at[c, pl.ds(base, CHUNK)], src_v)
        pltpu.sync_copy(dst_hbm.at[pl.ds(base, CHUNK)], dst_v)
        pltpu.async_copy(xcat_hbm.at[src_v], rows_v, sem).wait()
        pltpu.sync_copy(rows_v, agg_sh.at[dst_v], add=True)
        return carry

    lax.fori_loop(0, NCHUNK, chunk_body, 0)
    plsc.subcore_barrier()
    pltpu.sync_copy(agg_sh.at[stripe], out_hbm.at[c, stripe])


def _make_sc_agg():
    mesh = plsc.VectorSubcoreMesh(core_axis_name="c", subcore_axis_name="s")
    return functools.partial(
        pl.kernel,
        mesh=mesh,
        out_type=jax.ShapeDtypeStruct((NC, N_PAD, DH), jnp.float32),
        scratch_types=[
            pltpu.VMEM((CHUNK,), jnp.int32),
            pltpu.VMEM((CHUNK,), jnp.int32),
            pltpu.VMEM((CHUNK, DH), jnp.float32),
            pltpu.VMEM_SHARED((N_PAD, DH), jnp.float32),
            pltpu.SemaphoreType.DMA,
        ],
    )(_sc_agg_body)


_sc_agg = _make_sc_agg()


def _tc_mlp_body(x_ref, agg_ref, w_ref, b_ref, scale_ref, o_ref):
    agg = jnp.concatenate([agg_ref[0], agg_ref[1]], axis=1)
    h = x_ref[...] * scale_ref[0] + agg
    out = jnp.dot(h, w_ref[...], preferred_element_type=jnp.float32)
    o_ref[...] = jnp.maximum(out + b_ref[...], 0.0)


BM = 1000


def _tc_mlp(x, agg2, W, b2, scale):
    return pl.pallas_call(
        _tc_mlp_body,
        grid=(N_NODES // BM,),
        in_specs=[
            pl.BlockSpec((BM, D), lambda i: (i, 0)),
            pl.BlockSpec((NC, BM, DH), lambda i: (0, i, 0)),
            pl.BlockSpec((D, D), lambda i: (0, 0)),
            pl.BlockSpec((1, D), lambda i: (0, 0)),
            pl.BlockSpec(memory_space=pltpu.SMEM),
        ],
        out_specs=pl.BlockSpec((BM, D), lambda i: (i, 0)),
        out_shape=jax.ShapeDtypeStruct((N_NODES, D), jnp.float32),
    )(x, agg2, W, b2, scale)


def kernel(x, edge_index, W, b, eps):
    src = edge_index[0]
    dst = edge_index[1]
    pad = E_PAD - N_EDGES
    src_p = jnp.concatenate([src, jnp.full((pad,), ZROW, jnp.int32)])
    dst_p = jnp.concatenate([dst, jnp.zeros((pad,), jnp.int32)])
    half_rows = N_NODES + PAD_ROWS
    src2 = jnp.stack([src_p, src_p + half_rows])
    zpad = jnp.zeros((PAD_ROWS, DH), jnp.float32)
    xcat = jnp.concatenate([x[:, :DH], zpad, x[:, DH:], zpad], axis=0)
    zeros = jnp.zeros((ROWS_PER_TILE, DH), jnp.float32)

    agg2 = _sc_agg(xcat, src2, dst_p, zeros)

    scale = jnp.reshape(1.0 + eps, (1,)).astype(jnp.float32)
    b2 = jnp.reshape(b, (1, D))
    return _tc_mlp(x, agg2, W, b2, scale)


# speedup vs baseline: 2.8038x; 1.0725x over previous
import functools

import jax
import jax.numpy as jnp
from jax import lax
from jax.experimental import pallas as pl
from jax.experimental.pallas import tpu as pltpu
from jax.experimental.pallas import tpu_sc as plsc

N_NODES = 10000
N_EDGES = 160000
D = 256
DH = 128
NC = 2
NS = 16
PAD_ROWS = 8
ZROW = N_NODES
CHUNK = 128
NCHUNK = 80
E_PAD = NS * NCHUNK * CHUNK
EDGES_PER_TILE = E_PAD // NS
N_PAD = 10112
ROWS_PER_TILE = N_PAD // NS
NBUF = 2


def _sc_agg_body(xcat_hbm, src2_hbm, dst_hbm, zeros_hbm, out_hbm,
                 src_all, dsts, rows, agg_sh,
                 gsems, ssems, dsems, isem, zsem):
    c = lax.axis_index("c")
    s = lax.axis_index("s")
    stripe = pl.ds(s * ROWS_PER_TILE, ROWS_PER_TILE)
    zcopy = pltpu.async_copy(zeros_hbm, agg_sh.at[stripe], zsem)
    icopy = pltpu.async_copy(src2_hbm.at[c, s], src_all, isem)
    zcopy.wait()
    plsc.subcore_barrier()
    icopy.wait()

    def gather(k, j):
        return pltpu.async_copy(xcat_hbm.at[src_all.at[k]], rows.at[j],
                                gsems[j])

    def dst_copy(k, j):
        return pltpu.async_copy(dst_hbm.at[s, k], dsts.at[j], dsems[j])

    def outer_body(o, carry):
        for j in range(NBUF):
            k = o * NBUF + j
            dst_copy(k, j).wait()
            gather(k, j).wait()
            pltpu.async_copy(rows.at[j], agg_sh.at[dsts.at[j]], ssems[j],
                             add=True).wait()
        return carry

    lax.fori_loop(0, NCHUNK // NBUF, outer_body, 0)
    plsc.subcore_barrier()
    pltpu.sync_copy(agg_sh.at[stripe], out_hbm.at[c, stripe])


def _make_sc_agg():
    mesh = plsc.VectorSubcoreMesh(core_axis_name="c", subcore_axis_name="s")
    return functools.partial(
        pl.kernel,
        mesh=mesh,
        out_type=jax.ShapeDtypeStruct((NC, N_PAD, DH), jnp.float32),
        scratch_types=[
            pltpu.VMEM((NCHUNK, CHUNK), jnp.int32),
            pltpu.VMEM((NBUF, CHUNK), jnp.int32),
            pltpu.VMEM((NBUF, CHUNK, DH), jnp.float32),
            pltpu.VMEM_SHARED((N_PAD, DH), jnp.float32),
            [pltpu.SemaphoreType.DMA] * NBUF,
            [pltpu.SemaphoreType.DMA] * NBUF,
            [pltpu.SemaphoreType.DMA] * NBUF,
            pltpu.SemaphoreType.DMA,
            pltpu.SemaphoreType.DMA,
        ],
    )(_sc_agg_body)


_sc_agg = _make_sc_agg()


def _tc_mlp_body(x_ref, agg_ref, w_ref, b_ref, scale_ref, o_ref):
    agg = jnp.concatenate([agg_ref[0], agg_ref[1]], axis=1)
    h = x_ref[...] * scale_ref[0] + agg
    out = jnp.dot(h, w_ref[...], preferred_element_type=jnp.float32)
    o_ref[...] = jnp.maximum(out + b_ref[...], 0.0)


BM = 1000


def _tc_mlp(x, agg2, W, b2, scale):
    return pl.pallas_call(
        _tc_mlp_body,
        grid=(N_NODES // BM,),
        in_specs=[
            pl.BlockSpec((BM, D), lambda i: (i, 0)),
            pl.BlockSpec((NC, BM, DH), lambda i: (0, i, 0)),
            pl.BlockSpec((D, D), lambda i: (0, 0)),
            pl.BlockSpec((1, D), lambda i: (0, 0)),
            pl.BlockSpec(memory_space=pltpu.SMEM),
        ],
        out_specs=pl.BlockSpec((BM, D), lambda i: (i, 0)),
        out_shape=jax.ShapeDtypeStruct((N_NODES, D), jnp.float32),
    )(x, agg2, W, b2, scale)


def kernel(x, edge_index, W, b, eps):
    src = edge_index[0]
    dst = edge_index[1]
    pad = E_PAD - N_EDGES
    src_p = jnp.concatenate([src, jnp.full((pad,), ZROW, jnp.int32)])
    dst_p = jnp.concatenate([dst, jnp.zeros((pad,), jnp.int32)])
    half_rows = N_NODES + PAD_ROWS
    src2 = jnp.stack([src_p, src_p + half_rows])
    src2 = src2.reshape(NC, NS, NCHUNK, CHUNK)
    dst_p = dst_p.reshape(NS, NCHUNK, CHUNK)
    zpad = jnp.zeros((PAD_ROWS, DH), jnp.float32)
    xcat = jnp.concatenate([x[:, :DH], zpad, x[:, DH:], zpad], axis=0)
    zeros = jnp.zeros((ROWS_PER_TILE, DH), jnp.float32)

    agg2 = _sc_agg(xcat, src2, dst_p, zeros)

    scale = jnp.reshape(1.0 + eps, (1,)).astype(jnp.float32)
    b2 = jnp.reshape(b, (1, D))
    return _tc_mlp(x, agg2, W, b2, scale)


# speedup vs baseline: 3.5673x; 1.2723x over previous
import functools

import jax
import jax.numpy as jnp
from jax import lax
from jax.experimental import pallas as pl
from jax.experimental.pallas import tpu as pltpu
from jax.experimental.pallas import tpu_sc as plsc

N_NODES = 10000
N_EDGES = 160000
D = 256
DH = 128
NC = 2
NS = 16
PAD_ROWS = 8
ZROW = N_NODES
CHUNK = 128
NCHUNK = 80
E_PAD = NS * NCHUNK * CHUNK
EDGES_PER_TILE = E_PAD // NS
N_PAD = 10112
ROWS_PER_TILE = N_PAD // NS
NBUF = 2


def _sc_agg_body(xcat_hbm, src2_hbm, dst_hbm, zeros_hbm, out_hbm,
                 src_all, dsts, rows, agg_sh,
                 gsems, ssems, dsems, isem, zsem):
    c = lax.axis_index("c")
    s = lax.axis_index("s")
    stripe = pl.ds(s * ROWS_PER_TILE, ROWS_PER_TILE)
    zcopy = pltpu.async_copy(zeros_hbm, agg_sh.at[stripe], zsem)
    icopy = pltpu.async_copy(src2_hbm.at[c, s], src_all, isem)
    zcopy.wait()
    plsc.subcore_barrier()
    icopy.wait()

    def gather_start(k, j):
        pltpu.async_copy(xcat_hbm.at[src_all.at[k]], rows.at[j], gsems[j])

    def gather_wait(k, j):
        pltpu.make_async_copy(xcat_hbm.at[src_all.at[k]], rows.at[j],
                              gsems[j]).wait()

    def dst_start(k, j):
        pltpu.async_copy(dst_hbm.at[s, k], dsts.at[j], dsems[j])

    def dst_wait(k, j):
        pltpu.make_async_copy(dst_hbm.at[s, k], dsts.at[j], dsems[j]).wait()

    for j in range(NBUF):
        dst_start(j, j)
        gather_start(j, j)

    def outer_body(o, carry):
        for j in range(NBUF):
            k = o * NBUF + j
            gather_wait(k, j)
            dst_wait(k, j)
            pltpu.async_copy(rows.at[j], agg_sh.at[dsts.at[j]], ssems[j],
                             add=True).wait()

            @pl.when(k < NCHUNK - NBUF)
            def _():
                dst_start(k + NBUF, j)
                gather_start(k + NBUF, j)
        return carry

    lax.fori_loop(0, NCHUNK // NBUF, outer_body, 0)
    plsc.subcore_barrier()
    pltpu.sync_copy(agg_sh.at[stripe], out_hbm.at[c, stripe])


def _make_sc_agg():
    mesh = plsc.VectorSubcoreMesh(core_axis_name="c", subcore_axis_name="s")
    return functools.partial(
        pl.kernel,
        mesh=mesh,
        out_type=jax.ShapeDtypeStruct((NC, N_PAD, DH), jnp.float32),
        scratch_types=[
            pltpu.VMEM((NCHUNK, CHUNK), jnp.int32),
            pltpu.VMEM((NBUF, CHUNK), jnp.int32),
            pltpu.VMEM((NBUF, CHUNK, DH), jnp.float32),
            pltpu.VMEM_SHARED((N_PAD, DH), jnp.float32),
            [pltpu.SemaphoreType.DMA] * NBUF,
            [pltpu.SemaphoreType.DMA] * NBUF,
            [pltpu.SemaphoreType.DMA] * NBUF,
            pltpu.SemaphoreType.DMA,
            pltpu.SemaphoreType.DMA,
        ],
    )(_sc_agg_body)


_sc_agg = _make_sc_agg()


def _tc_mlp_body(x_ref, agg_ref, w_ref, b_ref, scale_ref, o_ref):
    agg = jnp.concatenate([agg_ref[0], agg_ref[1]], axis=1)
    h = x_ref[...] * scale_ref[0] + agg
    out = jnp.dot(h, w_ref[...], preferred_element_type=jnp.float32)
    o_ref[...] = jnp.maximum(out + b_ref[...], 0.0)


BM = 1000


def _tc_mlp(x, agg2, W, b2, scale):
    return pl.pallas_call(
        _tc_mlp_body,
        grid=(N_NODES // BM,),
        in_specs=[
            pl.BlockSpec((BM, D), lambda i: (i, 0)),
            pl.BlockSpec((NC, BM, DH), lambda i: (0, i, 0)),
            pl.BlockSpec((D, D), lambda i: (0, 0)),
            pl.BlockSpec((1, D), lambda i: (0, 0)),
            pl.BlockSpec(memory_space=pltpu.SMEM),
        ],
        out_specs=pl.BlockSpec((BM, D), lambda i: (i, 0)),
        out_shape=jax.ShapeDtypeStruct((N_NODES, D), jnp.float32),
    )(x, agg2, W, b2, scale)


def kernel(x, edge_index, W, b, eps):
    src = edge_index[0]
    dst = edge_index[1]
    pad = E_PAD - N_EDGES
    src_p = jnp.concatenate([src, jnp.full((pad,), ZROW, jnp.int32)])
    dst_p = jnp.concatenate([dst, jnp.zeros((pad,), jnp.int32)])
    half_rows = N_NODES + PAD_ROWS
    src2 = jnp.stack([src_p, src_p + half_rows])
    src2 = src2.reshape(NC, NS, NCHUNK, CHUNK)
    dst_p = dst_p.reshape(NS, NCHUNK, CHUNK)
    zpad = jnp.zeros((PAD_ROWS, DH), jnp.float32)
    xcat = jnp.concatenate([x[:, :DH], zpad, x[:, DH:], zpad], axis=0)
    zeros = jnp.zeros((ROWS_PER_TILE, DH), jnp.float32)

    agg2 = _sc_agg(xcat, src2, dst_p, zeros)

    scale = jnp.reshape(1.0 + eps, (1,)).astype(jnp.float32)
    b2 = jnp.reshape(b, (1, D))
    return _tc_mlp(x, agg2, W, b2, scale)


# speedup vs baseline: 3.5695x; 1.0006x over previous
import functools

import jax
import jax.numpy as jnp
from jax import lax
from jax.experimental import pallas as pl
from jax.experimental.pallas import tpu as pltpu
from jax.experimental.pallas import tpu_sc as plsc

N_NODES = 10000
N_EDGES = 160000
D = 256
DH = 128
NC = 2
NS = 16
PAD_ROWS = 8
ZROW = N_NODES
CHUNK = 128
NCHUNK = 80
E_PAD = NS * NCHUNK * CHUNK
EDGES_PER_TILE = E_PAD // NS
N_PAD = 10112
ROWS_PER_TILE = N_PAD // NS
NBUF = 2


def _sc_agg_body(xcat_hbm, src2_hbm, dst_hbm, zeros_hbm, out_hbm,
                 src_all, dsts, rows, agg_sh,
                 gsems, ssems, dsems, isem, zsem):
    c = lax.axis_index("c")
    s = lax.axis_index("s")
    stripe = pl.ds(s * ROWS_PER_TILE, ROWS_PER_TILE)
    zcopy = pltpu.async_copy(zeros_hbm, agg_sh.at[stripe], zsem)
    icopy = pltpu.async_copy(src2_hbm.at[c, s], src_all, isem)
    zcopy.wait()
    plsc.subcore_barrier()
    icopy.wait()

    HC = CHUNK // 2

    def gather_start(k, j):
        pltpu.async_copy(xcat_hbm.at[src_all.at[k, pl.ds(0, HC)]],
                         rows.at[j, pl.ds(0, HC)], gsems[j])
        pltpu.async_copy(xcat_hbm.at[src_all.at[k, pl.ds(HC, HC)]],
                         rows.at[j, pl.ds(HC, HC)], gsems[j])

    def gather_wait(k, j):
        pltpu.make_async_copy(xcat_hbm.at[src_all.at[k, pl.ds(0, HC)]],
                              rows.at[j, pl.ds(0, HC)], gsems[j]).wait()
        pltpu.make_async_copy(xcat_hbm.at[src_all.at[k, pl.ds(HC, HC)]],
                              rows.at[j, pl.ds(HC, HC)], gsems[j]).wait()

    def dst_start(k, j):
        pltpu.async_copy(dst_hbm.at[s, k], dsts.at[j], dsems[j])

    def dst_wait(k, j):
        pltpu.make_async_copy(dst_hbm.at[s, k], dsts.at[j], dsems[j]).wait()

    for j in range(NBUF):
        dst_start(j, j)
        gather_start(j, j)

    def outer_body(o, carry):
        for j in range(NBUF):
            k = o * NBUF + j
            gather_wait(k, j)
            dst_wait(k, j)
            pltpu.async_copy(rows.at[j], agg_sh.at[dsts.at[j]], ssems[j],
                             add=True).wait()

            @pl.when(k < NCHUNK - NBUF)
            def _():
                dst_start(k + NBUF, j)
                gather_start(k + NBUF, j)
        return carry

    lax.fori_loop(0, NCHUNK // NBUF, outer_body, 0)
    plsc.subcore_barrier()
    pltpu.sync_copy(agg_sh.at[stripe], out_hbm.at[c, stripe])


def _make_sc_agg():
    mesh = plsc.VectorSubcoreMesh(core_axis_name="c", subcore_axis_name="s")
    return functools.partial(
        pl.kernel,
        mesh=mesh,
        out_type=jax.ShapeDtypeStruct((NC, N_PAD, DH), jnp.float32),
        scratch_types=[
            pltpu.VMEM((NCHUNK, CHUNK), jnp.int32),
            pltpu.VMEM((NBUF, CHUNK), jnp.int32),
            pltpu.VMEM((NBUF, CHUNK, DH), jnp.float32),
            pltpu.VMEM_SHARED((N_PAD, DH), jnp.float32),
            [pltpu.SemaphoreType.DMA] * NBUF,
            [pltpu.SemaphoreType.DMA] * NBUF,
            [pltpu.SemaphoreType.DMA] * NBUF,
            pltpu.SemaphoreType.DMA,
            pltpu.SemaphoreType.DMA,
        ],
    )(_sc_agg_body)


_sc_agg = _make_sc_agg()


def _tc_mlp_body(x_ref, agg_ref, w_ref, b_ref, scale_ref, o_ref):
    agg = jnp.concatenate([agg_ref[0], agg_ref[1]], axis=1)
    h = x_ref[...] * scale_ref[0] + agg
    out = jnp.dot(h, w_ref[...], preferred_element_type=jnp.float32)
    o_ref[...] = jnp.maximum(out + b_ref[...], 0.0)


BM = 1000


def _tc_mlp(x, agg2, W, b2, scale):
    return pl.pallas_call(
        _tc_mlp_body,
        grid=(N_NODES // BM,),
        in_specs=[
            pl.BlockSpec((BM, D), lambda i: (i, 0)),
            pl.BlockSpec((NC, BM, DH), lambda i: (0, i, 0)),
            pl.BlockSpec((D, D), lambda i: (0, 0)),
            pl.BlockSpec((1, D), lambda i: (0, 0)),
            pl.BlockSpec(memory_space=pltpu.SMEM),
        ],
        out_specs=pl.BlockSpec((BM, D), lambda i: (i, 0)),
        out_shape=jax.ShapeDtypeStruct((N_NODES, D), jnp.float32),
    )(x, agg2, W, b2, scale)


def kernel(x, edge_index, W, b, eps):
    src = edge_index[0]
    dst = edge_index[1]
    pad = E_PAD - N_EDGES
    src_p = jnp.concatenate([src, jnp.full((pad,), ZROW, jnp.int32)])
    dst_p = jnp.concatenate([dst, jnp.zeros((pad,), jnp.int32)])
    half_rows = N_NODES + PAD_ROWS
    src2 = jnp.stack([src_p, src_p + half_rows])
    src2 = src2.reshape(NC, NS, NCHUNK, CHUNK)
    dst_p = dst_p.reshape(NS, NCHUNK, CHUNK)
    zpad = jnp.zeros((PAD_ROWS, DH), jnp.float32)
    xcat = jnp.concatenate([x[:, :DH], zpad, x[:, DH:], zpad], axis=0)
    zeros = jnp.zeros((ROWS_PER_TILE, DH), jnp.float32)

    agg2 = _sc_agg(xcat, src2, dst_p, zeros)

    scale = jnp.reshape(1.0 + eps, (1,)).astype(jnp.float32)
    b2 = jnp.reshape(b, (1, D))
    return _tc_mlp(x, agg2, W, b2, scale)


# speedup vs baseline: 4.4208x; 1.2385x over previous
import functools

import jax
import jax.numpy as jnp
from jax import lax
from jax.experimental import pallas as pl
from jax.experimental.pallas import tpu as pltpu
from jax.experimental.pallas import tpu_sc as plsc

N_NODES = 10000
N_EDGES = 160000
D = 256
DH = 128
NC = 2
NS = 16
PAD_ROWS = 8
ZROW = N_NODES
CHUNK = 120
NCHUNK = 85
E_PAD = NS * NCHUNK * CHUNK
EDGES_PER_TILE = E_PAD // NS
N_PAD = 10112
ROWS_PER_TILE = N_PAD // NS
NBUF = 3
IR = 6


def _sc_agg_body(xcat_hbm, src2_hbm, dst_hbm, zeros_hbm, out_hbm,
                 srcs, dsts, rows, agg_sh,
                 gsems, ssems, isems, dsems, zsem):
    c = lax.axis_index("c")
    s = lax.axis_index("s")
    stripe = pl.ds(s * ROWS_PER_TILE, ROWS_PER_TILE)
    zcopy = pltpu.async_copy(zeros_hbm, agg_sh.at[stripe], zsem)

    def idx_start(k, q):
        pltpu.async_copy(src2_hbm.at[c, s, k], srcs.at[q], isems[q])
        pltpu.async_copy(dst_hbm.at[s, k], dsts.at[q], dsems[q])

    def gather_start(k, j, q):
        pltpu.make_async_copy(src2_hbm.at[c, s, k], srcs.at[q],
                              isems[q]).wait()
        pltpu.async_copy(xcat_hbm.at[srcs.at[q]], rows.at[j], gsems[j])

    def gather_wait(k, j):
        pltpu.make_async_copy(xcat_hbm.at[srcs.at[0]], rows.at[j],
                              gsems[j]).wait()

    def scatter_start(k, j, q):
        pltpu.make_async_copy(dst_hbm.at[s, k], dsts.at[q],
                              dsems[q]).wait()
        pltpu.async_copy(rows.at[j], agg_sh.at[dsts.at[q]], ssems[j],
                         add=True)

    def scatter_wait(j):
        pltpu.make_async_copy(rows.at[0], agg_sh.at[dsts.at[0]],
                              ssems[j]).wait()

    for q in range(IR - 1):
        idx_start(q, q)
    zcopy.wait()
    plsc.subcore_barrier()
    gather_start(0, 0, 0)
    gather_start(1, 1, 1)

    gather_wait(0, 0)
    scatter_start(0, 0, 0)
    idx_start(IR - 1, IR - 1)
    gather_start(2, 2, 2)

    def outer_body(o, carry):
        for m in range(6):
            k = 1 + o * 6 + m
            j = (1 + m) % NBUF
            jp = m % NBUF
            q = (1 + m) % IR
            qn = (3 + m) % IR
            gather_wait(k, j)
            scatter_start(k, j, q)
            scatter_wait(jp)

            @pl.when(k <= NCHUNK - 6)
            def _():
                idx_start(k + 5, m % IR)

            @pl.when(k <= NCHUNK - 3)
            def _():
                gather_start(k + 2, jp, qn)
        return carry

    lax.fori_loop(0, (NCHUNK - 1) // 6, outer_body, 0)
    scatter_wait((NCHUNK - 1) % NBUF)
    plsc.subcore_barrier()
    pltpu.sync_copy(agg_sh.at[stripe], out_hbm.at[c, stripe])


def _make_sc_agg():
    mesh = plsc.VectorSubcoreMesh(core_axis_name="c", subcore_axis_name="s")
    return functools.partial(
        pl.kernel,
        mesh=mesh,
        out_type=jax.ShapeDtypeStruct((NC, N_PAD, DH), jnp.float32),
        scratch_types=[
            pltpu.VMEM((IR, CHUNK), jnp.int32),
            pltpu.VMEM((IR, CHUNK), jnp.int32),
            pltpu.VMEM((NBUF, CHUNK, DH), jnp.float32),
            pltpu.VMEM_SHARED((N_PAD, DH), jnp.float32),
            [pltpu.SemaphoreType.DMA] * NBUF,
            [pltpu.SemaphoreType.DMA] * NBUF,
            [pltpu.SemaphoreType.DMA] * IR,
            [pltpu.SemaphoreType.DMA] * IR,
            pltpu.SemaphoreType.DMA,
        ],
    )(_sc_agg_body)


_sc_agg = _make_sc_agg()


def _tc_mlp_body(x_ref, agg_ref, w_ref, b_ref, scale_ref, o_ref):
    agg = jnp.concatenate([agg_ref[0], agg_ref[1]], axis=1)
    h = x_ref[...] * scale_ref[0] + agg
    out = jnp.dot(h, w_ref[...], preferred_element_type=jnp.float32)
    o_ref[...] = jnp.maximum(out + b_ref[...], 0.0)


BM = 1000


def _tc_mlp(x, agg2, W, b2, scale):
    return pl.pallas_call(
        _tc_mlp_body,
        grid=(N_NODES // BM,),
        in_specs=[
            pl.BlockSpec((BM, D), lambda i: (i, 0)),
            pl.BlockSpec((NC, BM, DH), lambda i: (0, i, 0)),
            pl.BlockSpec((D, D), lambda i: (0, 0)),
            pl.BlockSpec((1, D), lambda i: (0, 0)),
            pl.BlockSpec(memory_space=pltpu.SMEM),
        ],
        out_specs=pl.BlockSpec((BM, D), lambda i: (i, 0)),
        out_shape=jax.ShapeDtypeStruct((N_NODES, D), jnp.float32),
    )(x, agg2, W, b2, scale)


def kernel(x, edge_index, W, b, eps):
    src = edge_index[0]
    dst = edge_index[1]
    pad = E_PAD - N_EDGES
    src_p = jnp.concatenate([src, jnp.full((pad,), ZROW, jnp.int32)])
    dst_p = jnp.concatenate([dst, jnp.zeros((pad,), jnp.int32)])
    half_rows = N_NODES + PAD_ROWS
    src2 = jnp.stack([src_p, src_p + half_rows])
    src2 = src2.reshape(NC, NS, NCHUNK, CHUNK)
    dst_p = dst_p.reshape(NS, NCHUNK, CHUNK)
    zpad = jnp.zeros((PAD_ROWS, DH), jnp.float32)
    xcat = jnp.concatenate([x[:, :DH], zpad, x[:, DH:], zpad], axis=0)
    zeros = jnp.zeros((ROWS_PER_TILE, DH), jnp.float32)

    agg2 = _sc_agg(xcat, src2, dst_p, zeros)

    scale = jnp.reshape(1.0 + eps, (1,)).astype(jnp.float32)
    b2 = jnp.reshape(b, (1, D))
    return _tc_mlp(x, agg2, W, b2, scale)
